# baseline (speedup 1.0000x reference)
import jax
import jax.numpy as jnp
from jax import lax
from jax.experimental import pallas as pl
from jax.experimental.pallas import tpu as pltpu
from jax.experimental.pallas import tpu_sc as plsc

N = 10000
E = 320000
F_IN = 128
HID = 128
C = 16
K_PROP = 10
ALPHA = 0.1

NS = 16
CH = 128
CPT = 160
EPT = CPT * CH
NP = 10240
ROWS = NP // NS


def _mlp_body(x_ref, w1_ref, b1_ref, w2_ref, b2_ref, z_ref):
    h = jnp.dot(x_ref[:], w1_ref[:], preferred_element_type=jnp.float32)
    h = jnp.maximum(h + b1_ref[:], 0.0)
    z = jnp.dot(h, w2_ref[:], preferred_element_type=jnp.float32)
    z_ref[:] = z + b2_ref[:]


def _mlp(x, W1, b1, W2, b2):
    return pl.pallas_call(
        _mlp_body,
        out_shape=jax.ShapeDtypeStruct((N, C), jnp.float32),
    )(x, W1, b1.reshape(1, HID), W2, b2.reshape(1, C))


def _final_body(o_ref, ls_ref, sm_ref):
    o = o_ref[:]
    m1 = jnp.max(o, axis=1, keepdims=True)
    e1 = jnp.exp(o - m1)
    ls_ref[:] = (o - m1) - jnp.log(jnp.sum(e1, axis=1, keepdims=True))
    m0 = jnp.max(o, axis=0, keepdims=True)
    e0 = jnp.exp(o - m0)
    sm_ref[:] = e0 / jnp.sum(e0, axis=0, keepdims=True)


def _finalize(out):
    return pl.pallas_call(
        _final_body,
        out_shape=(
            jax.ShapeDtypeStruct((N, C), jnp.float32),
            jax.ShapeDtypeStruct((N, C), jnp.float32),
        ),
    )(out)


def _deg_body(dst_hbm, deg_hbm, s_sh, dst_t, zero_t, ones_t,
              t0, t1, t2, t3, t4, t5, t6, t7, t8, t9):
    sems = (t0, t1, t2, t3, t4, t5, t6, t7, t8, t9)
    tid = lax.axis_index("s")
    row0 = tid * ROWS

    pltpu.sync_copy(dst_hbm.at[tid], dst_t)

    zeros16 = jnp.zeros((16,), jnp.float32)
    ones16 = jnp.ones((16,), jnp.float32)

    def _memset_zero(i, _):
        zero_t[i, :] = zeros16
        return 0
    lax.fori_loop(0, ROWS // 4, _memset_zero, 0)

    def _memset_one(i, _):
        ones_t[i, :] = ones16
        return 0
    lax.fori_loop(0, CH, _memset_one, 0)

    for q in range(4):
        pltpu.sync_copy(zero_t, s_sh.at[pl.ds(row0 + q * (ROWS // 4),
                                              ROWS // 4)])
    plsc.subcore_barrier()

    def _dgrp(g, _c):
        j0 = NBUF * g
        for r in range(NBUF):
            @pl.when(j0 >= NBUF)
            def _(r=r):
                pltpu.make_async_copy(ones_t, s_sh.at[dst_t.at[0]],
                                      sems[r]).wait()
            pltpu.async_copy(ones_t, s_sh.at[dst_t.at[j0 + r]], sems[r],
                             add=True)
        return 0
    lax.fori_loop(0, CPT // NBUF, _dgrp, 0)
    for r in range(NBUF):
        pltpu.make_async_copy(ones_t, s_sh.at[dst_t.at[0]], sems[r]).wait()

    plsc.subcore_barrier()
    pltpu.sync_copy(s_sh.at[pl.ds(row0, ROWS)],
                    deg_hbm.at[pl.ds(row0, ROWS)])


def _degrees(dst_p):
    mesh = plsc.VectorSubcoreMesh(
        core_axis_name="c", subcore_axis_name="s", num_cores=1)
    f = pl.kernel(
        _deg_body,
        out_type=(jax.ShapeDtypeStruct((NP, C), jnp.float32),),
        mesh=mesh,
        compiler_params=pltpu.CompilerParams(use_tc_tiling_on_sc=False),
        scratch_types=(
            pltpu.VMEM_SHARED((NP, C), jnp.float32),
            pltpu.VMEM((CPT, CH), jnp.int32),
            pltpu.VMEM((ROWS // 4, C), jnp.float32),
            pltpu.VMEM((CH, C), jnp.float32),
        ) + tuple(pltpu.SemaphoreType.DMA for _ in range(10)),
    )
    (deg16,) = f(dst_p)
    return deg16


NBUF = 10


def _sc_body(z_hbm, src_hbm, dst_hbm, deg_hbm,
             out_hbm,
             s_sh, u_sh,
             src_t, dst_t,
             m0, m1, m2, m3, m4, m5, m6, m7, m8, m9,
             a_t, c_t, u_t, s_t,
             zero_t,
             g0, g1, g2, g3, g4, g5, g6, g7, g8, g9,
             t0, t1, t2, t3, t4, t5, t6, t7, t8, t9):
    msgs = (m0, m1, m2, m3, m4, m5, m6, m7, m8, m9)
    semg = (g0, g1, g2, g3, g4, g5, g6, g7, g8, g9)
    sems = (t0, t1, t2, t3, t4, t5, t6, t7, t8, t9)
    tid = lax.axis_index("s")
    row0 = tid * ROWS

    pltpu.sync_copy(src_hbm.at[tid], src_t)
    pltpu.sync_copy(dst_hbm.at[tid], dst_t)
    pltpu.sync_copy(z_hbm.at[pl.ds(row0, ROWS)], u_t)

    zeros16 = jnp.zeros((16,), jnp.float32)

    def _memset_zero(i, _):
        zero_t[i, :] = zeros16
        return 0
    lax.fori_loop(0, ROWS // 4, _memset_zero, 0)

    for q in range(4):
        pltpu.sync_copy(zero_t, s_sh.at[pl.ds(row0 + q * (ROWS // 4),
                                              ROWS // 4)])

    pltpu.sync_copy(deg_hbm.at[pl.ds(row0, ROWS)], s_t)

    def _coef(i, _):
        deg = s_t[i, :] + 1.0
        bits = lax.bitcast_convert_type(deg, jnp.int32)
        bits = jnp.int32(0x5F3759DF) - lax.shift_right_logical(bits, 1)
        y = lax.bitcast_convert_type(bits, jnp.float32)
        y = y * (1.5 - 0.5 * deg * y * y)
        y = y * (1.5 - 0.5 * deg * y * y)
        y = y * (1.5 - 0.5 * deg * y * y)
        zrow = u_t[i, :]
        a_t[i, :] = 0.9 * y * y
        c_t[i, :] = 0.1 * y * zrow
        u_t[i, :] = y * zrow
        return 0
    lax.fori_loop(0, ROWS, _coef, 0)

    pltpu.sync_copy(u_t, u_sh.at[pl.ds(row0, ROWS)])
    plsc.subcore_barrier()

    def _step(k, _):
        for r in range(NBUF):
            pltpu.async_copy(u_sh.at[src_t.at[r]], msgs[r], semg[r])

        def _grp(g, _c):
            j0 = NBUF * g
            for r in range(NBUF):
                j = j0 + r
                pltpu.make_async_copy(u_sh.at[src_t.at[0]], msgs[r],
                                      semg[r]).wait()
                pltpu.async_copy(msgs[r], s_sh.at[dst_t.at[j]], sems[r],
                                 add=True)
                rp = (r + NBUF - 1) % NBUF
                jp = j - 1

                @pl.when((jp >= 0) & (jp + NBUF < CPT))
                def _(rp=rp, jp=jp):
                    pltpu.make_async_copy(msgs[rp], s_sh.at[dst_t.at[0]],
                                          sems[rp]).wait()
                    pltpu.async_copy(u_sh.at[src_t.at[jp + NBUF]], msgs[rp],
                                     semg[rp])
            return 0
        lax.fori_loop(0, CPT // NBUF, _grp, 0)
        for r in range(NBUF):
            pltpu.make_async_copy(msgs[r], s_sh.at[dst_t.at[0]],
                                  sems[r]).wait()

        plsc.subcore_barrier()

        pltpu.sync_copy(s_sh.at[pl.ds(row0, ROWS)], s_t)

        def _upd(q, _u):
            i = 4 * q
            u_t[i, :] = a_t[i, :] * (s_t[i, :] + u_t[i, :]) + c_t[i, :]
            u_t[i + 1, :] = (a_t[i + 1, :] * (s_t[i + 1, :] + u_t[i + 1, :])
                             + c_t[i + 1, :])
            u_t[i + 2, :] = (a_t[i + 2, :] * (s_t[i + 2, :] + u_t[i + 2, :])
                             + c_t[i + 2, :])
            u_t[i + 3, :] = (a_t[i + 3, :] * (s_t[i + 3, :] + u_t[i + 3, :])
                             + c_t[i + 3, :])
            return 0
        lax.fori_loop(0, ROWS // 4, _upd, 0)

        for q in range(4):
            pltpu.sync_copy(zero_t, s_sh.at[pl.ds(row0 + q * (ROWS // 4),
                                                  ROWS // 4)])
        pltpu.sync_copy(u_t, u_sh.at[pl.ds(row0, ROWS)])
        plsc.subcore_barrier()
        return 0
    lax.fori_loop(0, K_PROP, _step, 0)

    def _fin(i, _):
        deg = 0.9 / a_t[i, :]
        bits = lax.bitcast_convert_type(deg, jnp.int32)
        bits = jnp.int32(0x5F3759DF) - lax.shift_right_logical(bits, 1)
        y = lax.bitcast_convert_type(bits, jnp.float32)
        y = y * (1.5 - 0.5 * deg * y * y)
        y = y * (1.5 - 0.5 * deg * y * y)
        y = y * (1.5 - 0.5 * deg * y * y)
        s_t[i, :] = u_t[i, :] * deg * y
        return 0
    lax.fori_loop(0, ROWS, _fin, 0)
    pltpu.sync_copy(s_t, out_hbm.at[pl.ds(row0, ROWS)])


def _propagate(z, src_p, dst_p, deg16):
    mesh = plsc.VectorSubcoreMesh(
        core_axis_name="c", subcore_axis_name="s", num_cores=1)
    f = pl.kernel(
        _sc_body,
        out_type=(jax.ShapeDtypeStruct((NP, C), jnp.float32),),
        mesh=mesh,
        compiler_params=pltpu.CompilerParams(use_tc_tiling_on_sc=False),
        scratch_types=(
            pltpu.VMEM_SHARED((NP, C), jnp.float32),
            pltpu.VMEM_SHARED((NP, C), jnp.float32),
            pltpu.VMEM((CPT, CH), jnp.int32),
            pltpu.VMEM((CPT, CH), jnp.int32),
            pltpu.VMEM((CH, C), jnp.float32),
            pltpu.VMEM((CH, C), jnp.float32),
            pltpu.VMEM((CH, C), jnp.float32),
            pltpu.VMEM((CH, C), jnp.float32),
            pltpu.VMEM((CH, C), jnp.float32),
            pltpu.VMEM((CH, C), jnp.float32),
            pltpu.VMEM((CH, C), jnp.float32),
            pltpu.VMEM((CH, C), jnp.float32),
            pltpu.VMEM((CH, C), jnp.float32),
            pltpu.VMEM((CH, C), jnp.float32),
            pltpu.VMEM((ROWS, C), jnp.float32),
            pltpu.VMEM((ROWS, C), jnp.float32),
            pltpu.VMEM((ROWS, C), jnp.float32),
            pltpu.VMEM((ROWS, C), jnp.float32),
            pltpu.VMEM((ROWS // 4, C), jnp.float32),
        ) + tuple(pltpu.SemaphoreType.DMA for _ in range(20)),
    )
    (out,) = f(z, src_p, dst_p, deg16)
    return out[:N]


def kernel(x, edge_index, W1, b1, W2, b2):
    z = _mlp(x, W1, b1, W2, b2)
    z_p = jnp.concatenate([z, jnp.zeros((NP - N, C), jnp.float32)])

    src = edge_index[0]
    dst = edge_index[1]
    pad = NS * EPT - E
    padv = jnp.full((pad,), N, dtype=jnp.int32)
    src_p = jnp.concatenate([src, padv]).reshape(NS, CPT, CH)
    dst_p = jnp.concatenate([dst, padv]).reshape(NS, CPT, CH)

    deg16 = _degrees(dst_p)
    out = _propagate(z_p, src_p, dst_p, deg16)
    ls, sm = _finalize(out)
    return (ls, out, sm)


# speedup vs baseline: 1.0004x; 1.0004x over previous
import jax
import jax.numpy as jnp
from jax import lax
from jax.experimental import pallas as pl
from jax.experimental.pallas import tpu as pltpu
from jax.experimental.pallas import tpu_sc as plsc

N = 10000
E = 320000
F_IN = 128
HID = 128
C = 16
K_PROP = 10
ALPHA = 0.1

NS = 16
CH = 128
CPT = 160
EPT = CPT * CH
NP = 10240
ROWS = NP // NS


def _mlp_body(x_ref, w1_ref, b1_ref, w2_ref, b2_ref, z_ref):
    h = jnp.dot(x_ref[:], w1_ref[:], preferred_element_type=jnp.float32)
    h = jnp.maximum(h + b1_ref[:], 0.0)
    z = jnp.dot(h, w2_ref[:], preferred_element_type=jnp.float32)
    z_ref[:] = z + b2_ref[:]


def _mlp(x, W1, b1, W2, b2):
    return pl.pallas_call(
        _mlp_body,
        out_shape=jax.ShapeDtypeStruct((N, C), jnp.float32),
    )(x, W1, b1.reshape(1, HID), W2, b2.reshape(1, C))


def _final_body(o_ref, ls_ref, sm_ref):
    o = o_ref[:]
    m1 = jnp.max(o, axis=1, keepdims=True)
    e1 = jnp.exp(o - m1)
    ls_ref[:] = (o - m1) - jnp.log(jnp.sum(e1, axis=1, keepdims=True))
    m0 = jnp.max(o, axis=0, keepdims=True)
    e0 = jnp.exp(o - m0)
    sm_ref[:] = e0 / jnp.sum(e0, axis=0, keepdims=True)


def _finalize(out):
    return pl.pallas_call(
        _final_body,
        out_shape=(
            jax.ShapeDtypeStruct((N, C), jnp.float32),
            jax.ShapeDtypeStruct((N, C), jnp.float32),
        ),
    )(out)


def _deg_body(dst_hbm, deg_hbm, s_sh, dst_t, zero_t, ones_t,
              t0, t1, t2, t3, t4, t5, t6, t7):
    sems = (t0, t1, t2, t3, t4, t5, t6, t7)
    tid = lax.axis_index("s")
    row0 = tid * ROWS

    pltpu.sync_copy(dst_hbm.at[tid], dst_t)

    zeros16 = jnp.zeros((16,), jnp.float32)
    ones16 = jnp.ones((16,), jnp.float32)

    def _memset_zero(i, _):
        zero_t[i, :] = zeros16
        return 0
    lax.fori_loop(0, ROWS // 4, _memset_zero, 0)

    def _memset_one(i, _):
        ones_t[i, :] = ones16
        return 0
    lax.fori_loop(0, CH, _memset_one, 0)

    for q in range(4):
        pltpu.sync_copy(zero_t, s_sh.at[pl.ds(row0 + q * (ROWS // 4),
                                              ROWS // 4)])
    plsc.subcore_barrier()

    def _dgrp(g, _c):
        j0 = NBUF * g
        for r in range(NBUF):
            @pl.when(j0 >= NBUF)
            def _(r=r):
                pltpu.make_async_copy(ones_t, s_sh.at[dst_t.at[0]],
                                      sems[r]).wait()
            pltpu.async_copy(ones_t, s_sh.at[dst_t.at[j0 + r]], sems[r],
                             add=True)
        return 0
    lax.fori_loop(0, CPT // NBUF, _dgrp, 0)
    for r in range(NBUF):
        pltpu.make_async_copy(ones_t, s_sh.at[dst_t.at[0]], sems[r]).wait()

    plsc.subcore_barrier()
    pltpu.sync_copy(s_sh.at[pl.ds(row0, ROWS)],
                    deg_hbm.at[pl.ds(row0, ROWS)])


def _degrees(dst_p):
    mesh = plsc.VectorSubcoreMesh(
        core_axis_name="c", subcore_axis_name="s", num_cores=1)
    f = pl.kernel(
        _deg_body,
        out_type=(jax.ShapeDtypeStruct((NP, C), jnp.float32),),
        mesh=mesh,
        compiler_params=pltpu.CompilerParams(use_tc_tiling_on_sc=False),
        scratch_types=(
            pltpu.VMEM_SHARED((NP, C), jnp.float32),
            pltpu.VMEM((CPT, CH), jnp.int32),
            pltpu.VMEM((ROWS // 4, C), jnp.float32),
            pltpu.VMEM((CH, C), jnp.float32),
        ) + tuple(pltpu.SemaphoreType.DMA for _ in range(8)),
    )
    (deg16,) = f(dst_p)
    return deg16


NBUF = 8


def _sc_body(z_hbm, src_hbm, dst_hbm, deg_hbm,
             out_hbm,
             s_sh, u_sh,
             src_t, dst_t,
             m0, m1, m2, m3, m4, m5, m6, m7,
             a_t, c_t, u_t, s_t,
             zero_t,
             g0, g1, g2, g3, g4, g5, g6, g7,
             t0, t1, t2, t3, t4, t5, t6, t7):
    msgs = (m0, m1, m2, m3, m4, m5, m6, m7)
    semg = (g0, g1, g2, g3, g4, g5, g6, g7)
    sems = (t0, t1, t2, t3, t4, t5, t6, t7)
    tid = lax.axis_index("s")
    row0 = tid * ROWS

    pltpu.sync_copy(src_hbm.at[tid], src_t)
    pltpu.sync_copy(dst_hbm.at[tid], dst_t)
    pltpu.sync_copy(z_hbm.at[pl.ds(row0, ROWS)], u_t)

    zeros16 = jnp.zeros((16,), jnp.float32)

    def _memset_zero(i, _):
        zero_t[i, :] = zeros16
        return 0
    lax.fori_loop(0, ROWS // 4, _memset_zero, 0)

    for q in range(4):
        pltpu.sync_copy(zero_t, s_sh.at[pl.ds(row0 + q * (ROWS // 4),
                                              ROWS // 4)])

    pltpu.sync_copy(deg_hbm.at[pl.ds(row0, ROWS)], s_t)

    def _coef(i, _):
        deg = s_t[i, :] + 1.0
        bits = lax.bitcast_convert_type(deg, jnp.int32)
        bits = jnp.int32(0x5F3759DF) - lax.shift_right_logical(bits, 1)
        y = lax.bitcast_convert_type(bits, jnp.float32)
        y = y * (1.5 - 0.5 * deg * y * y)
        y = y * (1.5 - 0.5 * deg * y * y)
        y = y * (1.5 - 0.5 * deg * y * y)
        zrow = u_t[i, :]
        a_t[i, :] = 0.9 * y * y
        c_t[i, :] = 0.1 * y * zrow
        u_t[i, :] = y * zrow
        return 0
    lax.fori_loop(0, ROWS, _coef, 0)

    pltpu.sync_copy(u_t, u_sh.at[pl.ds(row0, ROWS)])
    plsc.subcore_barrier()

    def _step(k, _):
        for r in range(NBUF):
            pltpu.async_copy(u_sh.at[src_t.at[r]], msgs[r], semg[r])

        def _grp(g, _c):
            j0 = NBUF * g
            for r in range(NBUF):
                j = j0 + r
                pltpu.make_async_copy(u_sh.at[src_t.at[0]], msgs[r],
                                      semg[r]).wait()
                pltpu.async_copy(msgs[r], s_sh.at[dst_t.at[j]], sems[r],
                                 add=True)
                rp = (r + NBUF - 1) % NBUF
                jp = j - 1

                @pl.when((jp >= 0) & (jp + NBUF < CPT))
                def _(rp=rp, jp=jp):
                    pltpu.make_async_copy(msgs[rp], s_sh.at[dst_t.at[0]],
                                          sems[rp]).wait()
                    pltpu.async_copy(u_sh.at[src_t.at[jp + NBUF]], msgs[rp],
                                     semg[rp])
            return 0
        lax.fori_loop(0, CPT // NBUF, _grp, 0)
        for r in range(NBUF):
            pltpu.make_async_copy(msgs[r], s_sh.at[dst_t.at[0]],
                                  sems[r]).wait()

        plsc.subcore_barrier()

        pltpu.sync_copy(s_sh.at[pl.ds(row0, ROWS)], s_t)

        def _upd(q, _u):
            i = 4 * q
            u_t[i, :] = a_t[i, :] * (s_t[i, :] + u_t[i, :]) + c_t[i, :]
            u_t[i + 1, :] = (a_t[i + 1, :] * (s_t[i + 1, :] + u_t[i + 1, :])
                             + c_t[i + 1, :])
            u_t[i + 2, :] = (a_t[i + 2, :] * (s_t[i + 2, :] + u_t[i + 2, :])
                             + c_t[i + 2, :])
            u_t[i + 3, :] = (a_t[i + 3, :] * (s_t[i + 3, :] + u_t[i + 3, :])
                             + c_t[i + 3, :])
            return 0
        lax.fori_loop(0, ROWS // 4, _upd, 0)

        for q in range(4):
            pltpu.sync_copy(zero_t, s_sh.at[pl.ds(row0 + q * (ROWS // 4),
                                                  ROWS // 4)])
        pltpu.sync_copy(u_t, u_sh.at[pl.ds(row0, ROWS)])
        plsc.subcore_barrier()
        return 0
    lax.fori_loop(0, K_PROP, _step, 0)

    def _fin(i, _):
        deg = 0.9 / a_t[i, :]
        bits = lax.bitcast_convert_type(deg, jnp.int32)
        bits = jnp.int32(0x5F3759DF) - lax.shift_right_logical(bits, 1)
        y = lax.bitcast_convert_type(bits, jnp.float32)
        y = y * (1.5 - 0.5 * deg * y * y)
        y = y * (1.5 - 0.5 * deg * y * y)
        y = y * (1.5 - 0.5 * deg * y * y)
        s_t[i, :] = u_t[i, :] * deg * y
        return 0
    lax.fori_loop(0, ROWS, _fin, 0)
    pltpu.sync_copy(s_t, out_hbm.at[pl.ds(row0, ROWS)])


def _propagate(z, src_p, dst_p, deg16):
    mesh = plsc.VectorSubcoreMesh(
        core_axis_name="c", subcore_axis_name="s", num_cores=1)
    f = pl.kernel(
        _sc_body,
        out_type=(jax.ShapeDtypeStruct((NP, C), jnp.float32),),
        mesh=mesh,
        compiler_params=pltpu.CompilerParams(use_tc_tiling_on_sc=False),
        scratch_types=(
            pltpu.VMEM_SHARED((NP, C), jnp.float32),
            pltpu.VMEM_SHARED((NP, C), jnp.float32),
            pltpu.VMEM((CPT, CH), jnp.int32),
            pltpu.VMEM((CPT, CH), jnp.int32),
            pltpu.VMEM((CH, C), jnp.float32),
            pltpu.VMEM((CH, C), jnp.float32),
            pltpu.VMEM((CH, C), jnp.float32),
            pltpu.VMEM((CH, C), jnp.float32),
            pltpu.VMEM((CH, C), jnp.float32),
            pltpu.VMEM((CH, C), jnp.float32),
            pltpu.VMEM((CH, C), jnp.float32),
            pltpu.VMEM((CH, C), jnp.float32),
            pltpu.VMEM((ROWS, C), jnp.float32),
            pltpu.VMEM((ROWS, C), jnp.float32),
            pltpu.VMEM((ROWS, C), jnp.float32),
            pltpu.VMEM((ROWS, C), jnp.float32),
            pltpu.VMEM((ROWS // 4, C), jnp.float32),
        ) + tuple(pltpu.SemaphoreType.DMA for _ in range(16)),
    )
    (out,) = f(z, src_p, dst_p, deg16)
    return out[:N]


def kernel(x, edge_index, W1, b1, W2, b2):
    z = _mlp(x, W1, b1, W2, b2)
    z_p = jnp.concatenate([z, jnp.zeros((NP - N, C), jnp.float32)])

    src = edge_index[0]
    dst = edge_index[1]
    pad = NS * EPT - E
    padv = jnp.full((pad,), N, dtype=jnp.int32)
    src_p = jnp.concatenate([src, padv]).reshape(NS, CPT, CH)
    dst_p = jnp.concatenate([dst, padv]).reshape(NS, CPT, CH)

    deg16 = _degrees(dst_p)
    out = _propagate(z_p, src_p, dst_p, deg16)
    ls, sm = _finalize(out)
    return (ls, out, sm)
